# baseline (speedup 1.0000x reference)
import functools
import jax
import jax.numpy as jnp
from jax import lax
from jax.experimental import pallas as pl
from jax.experimental.pallas import tpu as pltpu, tpu_sc as plsc

BATCH = 4
SEQ_LEN = 8192
EMBED_DIM = 256

NC = 2
NS = 16
LANES = 16

NW = NC * NS
TOTAL = BATCH * SEQ_LEN * EMBED_DIM
POS_TOTAL = SEQ_LEN * EMBED_DIM
ELEMS_PER_W = TOTAL // NW
CHUNK = 16384
N_CHUNKS = ELEMS_PER_W // CHUNK
VECS_PER_CHUNK = CHUNK // LANES


def _body(in_hbm, pos_hbm, out_hbm, in_buf, acc_buf, sem_in, sem_pos, sem_out):
    wid = lax.axis_index("s") * NC + lax.axis_index("c")
    base = wid * ELEMS_PER_W
    pos_base = lax.rem(wid, 8) * ELEMS_PER_W

    def chunk_step(k, _):
        off = base + k * CHUNK
        pos_off = pos_base + k * CHUNK
        cp_pos = pltpu.make_async_copy(
            pos_hbm.at[pl.ds(pos_off, CHUNK)], acc_buf, sem_pos)
        cp_in = pltpu.make_async_copy(
            in_hbm.at[pl.ds(off, CHUNK)], in_buf, sem_in)
        cp_pos.start()
        cp_in.start()
        cp_pos.wait()
        cp_in.wait()

        def add_step(i, _):
            x = in_buf[pl.ds(i * LANES, LANES)]
            plsc.addupdate(acc_buf.at[pl.ds(i * LANES, LANES)], x)
            return 0

        lax.fori_loop(0, VECS_PER_CHUNK, add_step, 0, unroll=8)

        pltpu.make_async_copy(
            acc_buf, out_hbm.at[pl.ds(off, CHUNK)], sem_out).start()
        pltpu.make_async_copy(
            acc_buf, out_hbm.at[pl.ds(off, CHUNK)], sem_out).wait()
        return 0

    lax.fori_loop(0, N_CHUNKS, chunk_step, 0)


@jax.jit
def _pos_add(in_flat, pos_flat):
    mesh = plsc.VectorSubcoreMesh(core_axis_name="c", subcore_axis_name="s")
    return pl.kernel(
        _body,
        out_type=jax.ShapeDtypeStruct((TOTAL,), jnp.float32),
        mesh=mesh,
        scratch_types=[
            pltpu.VMEM((CHUNK,), jnp.float32),
            pltpu.VMEM((CHUNK,), jnp.float32),
            pltpu.SemaphoreType.DMA,
            pltpu.SemaphoreType.DMA,
            pltpu.SemaphoreType.DMA,
        ],
    )(in_flat, pos_flat)


def kernel(inputs, pos_table):
    out_flat = _pos_add(inputs.reshape(TOTAL), pos_table.reshape(POS_TOTAL))
    return out_flat.reshape(BATCH, SEQ_LEN, EMBED_DIM)


# speedup vs baseline: 1.2152x; 1.2152x over previous
import jax
import jax.numpy as jnp
from jax import lax
from jax.experimental import pallas as pl
from jax.experimental.pallas import tpu as pltpu, tpu_sc as plsc

BATCH = 4
SEQ_LEN = 8192
EMBED_DIM = 256

NC = 2
NS = 16
LANES = 16

NW = NC * NS
TOTAL = BATCH * SEQ_LEN * EMBED_DIM
POS_TOTAL = SEQ_LEN * EMBED_DIM
BATCH_STRIDE = SEQ_LEN * EMBED_DIM
POS_PER_W = POS_TOTAL // NW
CHUNK = 16384
CHUNKS_PER_BATCH = POS_PER_W // CHUNK
NBUF = 3
N_CHUNKS = BATCH * CHUNKS_PER_BATCH
VECS_PER_CHUNK = CHUNK // LANES


def _body(in_hbm, pos_hbm, out_hbm, pos_v, bufs, sem_pos, sems_in, sems_out):
    wid = lax.axis_index("s") * NC + lax.axis_index("c")
    pos_base = wid * POS_PER_W

    pltpu.make_async_copy(
        pos_hbm.at[pl.ds(pos_base, POS_PER_W)], pos_v, sem_pos).start()

    def hbm_off(k):
        b, piece = divmod(k, CHUNKS_PER_BATCH)
        return b * BATCH_STRIDE + pos_base + piece * CHUNK

    def in_cp(k, slot):
        return pltpu.make_async_copy(
            in_hbm.at[pl.ds(hbm_off(k), CHUNK)], bufs[slot], sems_in[slot])

    def out_cp(k, slot):
        return pltpu.make_async_copy(
            bufs[slot], out_hbm.at[pl.ds(hbm_off(k), CHUNK)], sems_out[slot])

    for k in range(NBUF - 1):
        in_cp(k, k % NBUF).start()

    pltpu.make_async_copy(
        pos_hbm.at[pl.ds(pos_base, POS_PER_W)], pos_v, sem_pos).wait()

    for k in range(N_CHUNKS):
        slot = k % NBUF
        nk = k + NBUF - 1
        if nk < N_CHUNKS:
            nslot = nk % NBUF
            if nk >= NBUF:
                out_cp(nk - NBUF, nslot).wait()
            in_cp(nk, nslot).start()
        in_cp(k, slot).wait()

        pos_off = (k % CHUNKS_PER_BATCH) * CHUNK
        buf = bufs[slot]

        def add_step(i, _, buf=buf, pos_off=pos_off):
            x = pos_v[pl.ds(pos_off + i * LANES, LANES)]
            plsc.addupdate(buf.at[pl.ds(i * LANES, LANES)], x)
            return 0

        lax.fori_loop(0, VECS_PER_CHUNK, add_step, 0, unroll=8)

        out_cp(k, slot).start()

    for k in range(N_CHUNKS - NBUF, N_CHUNKS):
        if k >= 0:
            out_cp(k, k % NBUF).wait()


@jax.jit
def _pos_add(in_flat, pos_flat):
    mesh = plsc.VectorSubcoreMesh(core_axis_name="c", subcore_axis_name="s")
    return pl.kernel(
        _body,
        out_type=jax.ShapeDtypeStruct((TOTAL,), jnp.float32),
        mesh=mesh,
        scratch_types=[
            pltpu.VMEM((POS_PER_W,), jnp.float32),
            [pltpu.VMEM((CHUNK,), jnp.float32) for _ in range(NBUF)],
            pltpu.SemaphoreType.DMA,
            [pltpu.SemaphoreType.DMA for _ in range(NBUF)],
            [pltpu.SemaphoreType.DMA for _ in range(NBUF)],
        ],
    )(in_flat, pos_flat)


def kernel(inputs, pos_table):
    out_flat = _pos_add(inputs.reshape(TOTAL), pos_table.reshape(POS_TOTAL))
    return out_flat.reshape(BATCH, SEQ_LEN, EMBED_DIM)


# speedup vs baseline: 1.6222x; 1.3349x over previous
import jax
import jax.numpy as jnp
from jax import lax
from jax.experimental import pallas as pl
from jax.experimental.pallas import tpu as pltpu, tpu_sc as plsc

BATCH = 4
SEQ_LEN = 8192
EMBED_DIM = 256

NC = 2
NS = 16
LANES = 16

NW = NC * NS
ROWS_PER_W = SEQ_LEN // NW
CHUNK_ROWS = 64
CHUNKS_PER_BATCH = ROWS_PER_W // CHUNK_ROWS
NBUF = 3
N_CHUNKS = BATCH * CHUNKS_PER_BATCH
VECS_PER_ROW = EMBED_DIM // LANES


def _body(in_hbm, pos_hbm, out_hbm, pos_v, bufs, sem_pos, sems_in, sems_out):
    wid = lax.axis_index("s") * NC + lax.axis_index("c")
    s_base = wid * ROWS_PER_W

    cp_pos = pltpu.make_async_copy(
        pos_hbm.at[pl.ds(s_base, ROWS_PER_W), :], pos_v, sem_pos)
    cp_pos.start()

    def in_cp(k, slot):
        b, piece = divmod(k, CHUNKS_PER_BATCH)
        s0 = s_base + piece * CHUNK_ROWS
        return pltpu.make_async_copy(
            in_hbm.at[b, pl.ds(s0, CHUNK_ROWS), :], bufs[slot], sems_in[slot])

    def out_cp(k, slot):
        b, piece = divmod(k, CHUNKS_PER_BATCH)
        s0 = s_base + piece * CHUNK_ROWS
        return pltpu.make_async_copy(
            bufs[slot], out_hbm.at[b, pl.ds(s0, CHUNK_ROWS), :], sems_out[slot])

    for k in range(NBUF - 1):
        in_cp(k, k % NBUF).start()

    cp_pos.wait()

    for k in range(N_CHUNKS):
        slot = k % NBUF
        nk = k + NBUF - 1
        if nk < N_CHUNKS:
            nslot = nk % NBUF
            if nk >= NBUF:
                out_cp(nk - NBUF, nslot).wait()
            in_cp(nk, nslot).start()
        in_cp(k, slot).wait()

        row0 = (k % CHUNKS_PER_BATCH) * CHUNK_ROWS
        buf = bufs[slot]

        def add_row(r, _, buf=buf, row0=row0):
            for c in range(VECS_PER_ROW):
                x = pos_v[row0 + r, pl.ds(c * LANES, LANES)]
                plsc.addupdate(buf.at[r, pl.ds(c * LANES, LANES)], x)
            return 0

        lax.fori_loop(0, CHUNK_ROWS, add_row, 0, unroll=2)

        out_cp(k, slot).start()

    for k in range(N_CHUNKS - NBUF, N_CHUNKS):
        if k >= 0:
            out_cp(k, k % NBUF).wait()


@jax.jit
def _pos_add(inputs, pos_table):
    mesh = plsc.VectorSubcoreMesh(core_axis_name="c", subcore_axis_name="s")
    return pl.kernel(
        _body,
        out_type=jax.ShapeDtypeStruct((BATCH, SEQ_LEN, EMBED_DIM), jnp.float32),
        mesh=mesh,
        scratch_types=[
            pltpu.VMEM((ROWS_PER_W, EMBED_DIM), jnp.float32),
            [pltpu.VMEM((CHUNK_ROWS, EMBED_DIM), jnp.float32)
             for _ in range(NBUF)],
            pltpu.SemaphoreType.DMA,
            [pltpu.SemaphoreType.DMA for _ in range(NBUF)],
            [pltpu.SemaphoreType.DMA for _ in range(NBUF)],
        ],
    )(inputs, pos_table)


def kernel(inputs, pos_table):
    return _pos_add(inputs, pos_table)
